# Initial kernel scaffold; baseline (speedup 1.0000x reference)
import jax
import jax.numpy as jnp
from jax.experimental import pallas as pl


def kernel(x, edge_index, batch, W1, b1, g1, bt1, W2, b2, g2, bt2, fcW, fcb):
    raise NotImplementedError("write your pallas kernel here")


# speedup vs baseline: 7.2487x; 7.2487x over previous
import functools

import jax
import jax.numpy as jnp
from jax import lax
from jax.experimental import pallas as pl
from jax.experimental.pallas import tpu as pltpu, tpu_sc as plsc

N = 10000
D = 128
E = 320000
NLAYERS = 5
G = 64
C = 10
EPS = 1e-5

NS = 16
EPT = E // NS
K = 125
CHUNKS = EPT // K
NBUF = 2
IB = 20
NBLK = CHUNKS // IB
ACC_N = 10240
RPT = ACC_N // NS
RSTG = 80


def _agg_body(h_hbm, src_hbm, dst_hbm, out_hbm,
              src_v, dst_v, rows_v, acc_sh, sem_g):
    sid = lax.axis_index("s")
    base = sid * RPT

    def zrow(r, carry):
        for c in range(D // 16):
            rows_v[0, r, pl.ds(c * 16, 16)] = jnp.zeros((16,), jnp.float32)
        return carry
    lax.fori_loop(0, RSTG, zrow, 0)
    stage = rows_v.at[0, pl.ds(0, RSTG)]
    for t in range(RPT // RSTG):
        pltpu.sync_copy(stage, acc_sh.at[pl.ds(base + t * RSTG, RSTG)])
    plsc.subcore_barrier()

    def gather_start(j, b):
        pltpu.make_async_copy(h_hbm.at[src_v.at[j]], rows_v.at[b], sem_g).start()

    def gather_wait(b):
        pltpu.make_async_copy(h_hbm.at[src_v.at[0]], rows_v.at[b], sem_g).wait()

    for g in range(NBLK):
        pltpu.sync_copy(src_hbm.at[sid * NBLK + g], src_v)
        pltpu.sync_copy(dst_hbm.at[sid * NBLK + g], dst_v)
        for b in range(NBUF):
            gather_start(b, b)

        def round_(j0, carry):
            for b in range(NBUF):
                j = j0 * NBUF + b
                gather_wait(b)
                pltpu.sync_copy(rows_v.at[b], acc_sh.at[dst_v.at[j]], add=True)
                @pl.when(j + NBUF < IB)
                def _():
                    gather_start(j + NBUF, b)
            return carry
        lax.fori_loop(0, IB // NBUF, round_, 0)

    plsc.subcore_barrier()

    for t in range(RPT // RSTG):
        pltpu.sync_copy(acc_sh.at[pl.ds(base + t * RSTG, RSTG)], stage)
        pltpu.sync_copy(stage, out_hbm.at[pl.ds(base + t * RSTG, RSTG)])


_agg = functools.partial(
    pl.kernel,
    out_type=jax.ShapeDtypeStruct((ACC_N, D), jnp.float32),
    mesh=plsc.VectorSubcoreMesh(core_axis_name="c", subcore_axis_name="s",
                                num_cores=1),
    scratch_types=[
        pltpu.VMEM((IB, K), jnp.int32),
        pltpu.VMEM((IB, K), jnp.int32),
        pltpu.VMEM((NBUF, K, D), jnp.float32),
        pltpu.VMEM_SHARED((ACC_N, D), jnp.float32),
        pltpu.SemaphoreType.DMA,
    ],
)(_agg_body)


def _mlp_body(h_ref, p_ref, w1, b1, g1, bt1, w2, b2, g2, bt2, o_ref):
    agg = lax.slice(p_ref[...], (0, 0), (N, D))
    m = h_ref[...] + agg
    y = jnp.dot(m, w1[...], preferred_element_type=jnp.float32) + b1[...]
    mu = jnp.mean(y, axis=0, keepdims=True)
    var = jnp.mean((y - mu) * (y - mu), axis=0, keepdims=True)
    y = g1[...] * (y - mu) * lax.rsqrt(var + EPS) + bt1[...]
    y = jnp.maximum(y, 0.0)
    z = jnp.dot(y, w2[...], preferred_element_type=jnp.float32) + b2[...]
    z = jnp.maximum(z, 0.0)
    mu2 = jnp.mean(z, axis=0, keepdims=True)
    var2 = jnp.mean((z - mu2) * (z - mu2), axis=0, keepdims=True)
    z = g2[...] * (z - mu2) * lax.rsqrt(var2 + EPS) + bt2[...]
    o_ref[...] = jnp.maximum(z, 0.0)


_mlp = pl.pallas_call(
    _mlp_body,
    out_shape=jax.ShapeDtypeStruct((N, D), jnp.float32),
)


def _pool_body(h_ref, batch_ref, fcw_ref, fcb_ref, o_ref):
    gids = lax.broadcasted_iota(jnp.int32, (N, 128), 1)
    onehot = (batch_ref[...] == gids).astype(jnp.float32)
    pooled = lax.dot_general(onehot, h_ref[...], (((0,), (0,)), ((), ())),
                             preferred_element_type=jnp.float32)
    out = jnp.dot(pooled, fcw_ref[...],
                  preferred_element_type=jnp.float32) + fcb_ref[...]
    o_ref[...] = out[:G, :]


_pool = pl.pallas_call(
    _pool_body,
    out_shape=jax.ShapeDtypeStruct((G, C), jnp.float32),
)


def kernel(x, edge_index, batch, W1, b1, g1, bt1, W2, b2, g2, bt2, fcW, fcb):
    src = edge_index[0].reshape(NS * NBLK, IB, K)
    dst = edge_index[1].reshape(NS * NBLK, IB, K)
    batch2d = batch.reshape(N, 1)
    h = x
    for i in range(NLAYERS):
        parts = _agg(h, src, dst)
        h = _mlp(h, parts,
                 W1[i], b1[i].reshape(1, D), g1[i].reshape(1, D),
                 bt1[i].reshape(1, D),
                 W2[i], b2[i].reshape(1, D), g2[i].reshape(1, D),
                 bt2[i].reshape(1, D))
    return _pool(h, batch2d, fcW, fcb.reshape(1, C))
